# baseline (speedup 1.0000x reference)
import functools
import math

import jax
import jax.numpy as jnp
from jax.experimental import pallas as pl
from jax.experimental.pallas import tpu as pltpu

DELTA_VAR = 0.5
DELTA_DIST = 2.0
ALPHA = 1.0
BETA = 1.0
GAMMA = 0.001
INSTANCE_W = 1.0
PMAPS_THRESHOLD = 0.9
TWO_SIGMA = DELTA_VAR * DELTA_VAR / -math.log(PMAPS_THRESHOLD)
NEG_INV_TS = -1.0 / TWO_SIGMA
NEG2 = NEG_INV_TS * math.log2(math.e)
C = 64
EPS = 1e-6

BLOCK = 65536

_DN_RHS_T = (((1,), (1,)), ((), ()))
_DN_MATMUL = (((1,), (0,)), ((), ()))


def _dot(a, b, dn):
    return jax.lax.dot_general(
        a, b, dn,
        preferred_element_type=jnp.float32,
        precision=jax.lax.Precision.DEFAULT)


def _loss_kernel(p_total,
                 emb_ref, tgt_ref, out_ref,
                 sumst_ref, cnt_ref, means_ref, meanst_ref, msc_ref,
                 invc_ref, seghinge_ref, inter_ref, p2_ref):
    p_id = pl.program_id(0)
    i_id = pl.program_id(1)
    nblocks = pl.num_programs(1)

    e = emb_ref[...]
    t = tgt_ref[...]
    b = e.shape[1]
    ids = jax.lax.broadcasted_iota(jnp.int32, (C, b), 0)
    ohb = (ids == t).astype(jnp.bfloat16)
    en2n2 = jnp.sum(e * e, axis=0, keepdims=True) * (2.0 * NEG2)
    eaug = jnp.concatenate(
        [e.astype(jnp.bfloat16),
         jnp.ones((1, b), jnp.bfloat16),
         en2n2.astype(jnp.bfloat16)], axis=0)

    @pl.when(jnp.logical_and(p_id == 0, i_id == 0))
    def _init():
        sumst_ref[...] = jnp.zeros_like(sumst_ref)
        seghinge_ref[...] = jnp.zeros_like(seghinge_ref)
        inter_ref[0, 0] = 0.0
        p2_ref[0, 0] = 0.0

    @pl.when(p_id == 0)
    def _pass0():
        sumst_ref[...] += _dot(eaug, ohb, _DN_RHS_T)

    @pl.when(jnp.logical_and(p_id == 1, i_id == 0))
    def _means():
        safe_row = jnp.maximum(sumst_ref[16:17, :], 1.0)
        invc_ref[...] = 1.0 / safe_row
        cnt_ref[...] = sumst_ref[16:17, :]
        meanst_ref[...] = sumst_ref[0:16, :] / safe_row
        ri = jax.lax.broadcasted_iota(jnp.int32, (C, C), 0)
        ci = jax.lax.broadcasted_iota(jnp.int32, (C, C), 1)
        eye = (ri == ci).astype(jnp.float32)
        mc = _dot(eye, meanst_ref[...], _DN_RHS_T)
        means_ref[...] = mc
        mun2 = jnp.sum(mc * mc, axis=1, keepdims=True)
        rows = jax.lax.broadcasted_iota(jnp.int32, (C, 1), 0)
        mun2n = jnp.where(rows == 0, -1e30, mun2 * (2.0 * NEG2))
        msc_ref[...] = jnp.concatenate(
            [mc * (-4.0 * NEG2), mun2n,
             jnp.ones((C, 1), jnp.float32)], axis=1).astype(jnp.bfloat16)

    @pl.when(p_id == 1)
    def _pass1():
        musel = _dot(meanst_ref[...].astype(jnp.bfloat16), ohb,
                     _DN_MATMUL)
        diff = e - musel
        d2sel = jnp.sum(diff * diff, axis=0, keepdims=True)
        hinge = jnp.maximum(jnp.sqrt(d2sel) - DELTA_VAR, 0.0) ** 2
        seghinge_ref[...] += _dot(hinge.astype(jnp.bfloat16), ohb,
                                  _DN_RHS_T)
        q = jnp.exp2(_dot(msc_ref[...], eaug, _DN_MATMUL))
        p2_ref[0, 0] += jnp.sum(q)
        pmsel = jnp.exp2(d2sel * NEG2)
        inter_ref[0, 0] += jnp.sum(jnp.where(t == 0, 0.0, pmsel))

    @pl.when(jnp.logical_and(p_id == 1, i_id == nblocks - 1))
    def _final():
        means = means_ref[...]
        gm = _dot(means, means, _DN_RHS_T)
        ri = jax.lax.broadcasted_iota(jnp.int32, (C, C), 0)
        ci = jax.lax.broadcasted_iota(jnp.int32, (C, C), 1)
        diag = jnp.where(ri == ci, gm, 0.0)
        mun2_row = jnp.sum(diag, axis=0, keepdims=True)
        mun2_col = jnp.sum(diag, axis=1, keepdims=True)
        dd2 = jnp.maximum(mun2_col + mun2_row - 2.0 * gm, 0.0)
        dmat = jnp.sqrt(dd2)
        hinged = jnp.where(
            ri == ci, 0.0,
            jnp.maximum(2.0 * DELTA_DIST - dmat, 0.0) ** 2)
        distance_term = jnp.sum(hinged) / (C * (C - 1))
        variance_term = jnp.sum(seghinge_ref[...] * invc_ref[...]) / C
        reg_term = jnp.sum(jnp.sqrt(mun2_row)) / C
        cols = jax.lax.broadcasted_iota(jnp.int32, (1, C), 1)
        count0 = jnp.sum(jnp.where(cols == 0, cnt_ref[...], 0.0))
        m2 = p_total - count0
        denom = jnp.maximum(p2_ref[0, 0] + m2, EPS)
        dice = 2.0 * inter_ref[0, 0] / denom
        instance_term = 1.0 - dice
        loss = (ALPHA * variance_term + BETA * distance_term
                + GAMMA * reg_term + INSTANCE_W * instance_term)
        out_ref[0, 0] = 2.0 * loss


@jax.jit
def _run(emb, tgt):
    p = emb.shape[1]
    nb = p // BLOCK
    out = pl.pallas_call(
        functools.partial(_loss_kernel, float(p)),
        grid=(2, nb),
        in_specs=[
            pl.BlockSpec((16, BLOCK), lambda pp, i: (0, i)),
            pl.BlockSpec((1, BLOCK), lambda pp, i: (0, i)),
        ],
        out_specs=pl.BlockSpec((1, 1), lambda pp, i: (0, 0),
                               memory_space=pltpu.SMEM),
        out_shape=jax.ShapeDtypeStruct((1, 1), jnp.float32),
        scratch_shapes=[
            pltpu.VMEM((18, C), jnp.float32),
            pltpu.VMEM((1, C), jnp.float32),
            pltpu.VMEM((C, 16), jnp.float32),
            pltpu.VMEM((16, C), jnp.float32),
            pltpu.VMEM((C, 18), jnp.bfloat16),
            pltpu.VMEM((1, C), jnp.float32),
            pltpu.VMEM((1, C), jnp.float32),
            pltpu.SMEM((1, 1), jnp.float32),
            pltpu.SMEM((1, 1), jnp.float32),
        ],
    )(emb, tgt)
    return out[0, 0]


def kernel(input_, target):
    n_batches = input_.shape[0]
    emb = input_[n_batches - 1].reshape(16, -1)
    tgt = target[n_batches - 1, 0].reshape(1, -1)
    return _run(emb, tgt) / n_batches


# speedup vs baseline: 1.0146x; 1.0146x over previous
import functools
import math

import jax
import jax.numpy as jnp
from jax.experimental import pallas as pl
from jax.experimental.pallas import tpu as pltpu

DELTA_VAR = 0.5
DELTA_DIST = 2.0
ALPHA = 1.0
BETA = 1.0
GAMMA = 0.001
INSTANCE_W = 1.0
PMAPS_THRESHOLD = 0.9
TWO_SIGMA = DELTA_VAR * DELTA_VAR / -math.log(PMAPS_THRESHOLD)
NEG_INV_TS = -1.0 / TWO_SIGMA
NEG2 = NEG_INV_TS * math.log2(math.e)
C = 64
EPS = 1e-6

BLOCK = 16384

_DN_RHS_T = (((1,), (1,)), ((), ()))
_DN_MATMUL = (((1,), (0,)), ((), ()))


def _dot(a, b, dn):
    return jax.lax.dot_general(
        a, b, dn,
        preferred_element_type=jnp.float32,
        precision=jax.lax.Precision.DEFAULT)


def _loss_kernel(p_total,
                 emb_ref, tgt_ref, out_ref,
                 sumst_ref, cnt_ref, means_ref, meanst_ref, msc_ref,
                 invc_ref, seghinge_ref, inter_ref, p2_ref):
    p_id = pl.program_id(0)
    i_id = pl.program_id(1)
    nblocks = pl.num_programs(1)

    e = emb_ref[...]
    t = tgt_ref[...]
    b = e.shape[1]
    ids = jax.lax.broadcasted_iota(jnp.int32, (C, b), 0)
    ohb = (ids == t).astype(jnp.bfloat16)
    en2n2 = jnp.sum(e * e, axis=0, keepdims=True) * (2.0 * NEG2)
    eaug = jnp.concatenate(
        [e.astype(jnp.bfloat16),
         jnp.ones((1, b), jnp.bfloat16),
         en2n2.astype(jnp.bfloat16)], axis=0)

    @pl.when(jnp.logical_and(p_id == 0, i_id == 0))
    def _init():
        sumst_ref[...] = jnp.zeros_like(sumst_ref)
        seghinge_ref[...] = jnp.zeros_like(seghinge_ref)
        inter_ref[0, 0] = 0.0
        p2_ref[0, 0] = 0.0

    @pl.when(p_id == 0)
    def _pass0():
        sumst_ref[...] += _dot(eaug, ohb, _DN_RHS_T)

    @pl.when(jnp.logical_and(p_id == 1, i_id == 0))
    def _means():
        safe_row = jnp.maximum(sumst_ref[16:17, :], 1.0)
        invc_ref[...] = 1.0 / safe_row
        cnt_ref[...] = sumst_ref[16:17, :]
        meanst_ref[...] = sumst_ref[0:16, :] / safe_row
        ri = jax.lax.broadcasted_iota(jnp.int32, (C, C), 0)
        ci = jax.lax.broadcasted_iota(jnp.int32, (C, C), 1)
        eye = (ri == ci).astype(jnp.float32)
        mc = _dot(eye, meanst_ref[...], _DN_RHS_T)
        means_ref[...] = mc
        mun2 = jnp.sum(mc * mc, axis=1, keepdims=True)
        rows = jax.lax.broadcasted_iota(jnp.int32, (C, 1), 0)
        mun2n = jnp.where(rows == 0, -1e30, mun2 * (2.0 * NEG2))
        msc_ref[...] = jnp.concatenate(
            [mc * (-4.0 * NEG2), mun2n,
             jnp.ones((C, 1), jnp.float32)], axis=1).astype(jnp.bfloat16)

    @pl.when(p_id == 1)
    def _pass1():
        musel = _dot(meanst_ref[...].astype(jnp.bfloat16), ohb,
                     _DN_MATMUL)
        diff = e - musel
        d2sel = jnp.sum(diff * diff, axis=0, keepdims=True)
        hinge = jnp.maximum(jnp.sqrt(d2sel) - DELTA_VAR, 0.0) ** 2
        seghinge_ref[...] += _dot(hinge.astype(jnp.bfloat16), ohb,
                                  _DN_RHS_T)
        q = jnp.exp2(_dot(msc_ref[...], eaug, _DN_MATMUL))
        p2_ref[0, 0] += jnp.sum(q)
        pmsel = jnp.exp2(d2sel * NEG2)
        inter_ref[0, 0] += jnp.sum(jnp.where(t == 0, 0.0, pmsel))

    @pl.when(jnp.logical_and(p_id == 1, i_id == nblocks - 1))
    def _final():
        means = means_ref[...]
        gm = _dot(means, means, _DN_RHS_T)
        ri = jax.lax.broadcasted_iota(jnp.int32, (C, C), 0)
        ci = jax.lax.broadcasted_iota(jnp.int32, (C, C), 1)
        diag = jnp.where(ri == ci, gm, 0.0)
        mun2_row = jnp.sum(diag, axis=0, keepdims=True)
        mun2_col = jnp.sum(diag, axis=1, keepdims=True)
        dd2 = jnp.maximum(mun2_col + mun2_row - 2.0 * gm, 0.0)
        dmat = jnp.sqrt(dd2)
        hinged = jnp.where(
            ri == ci, 0.0,
            jnp.maximum(2.0 * DELTA_DIST - dmat, 0.0) ** 2)
        distance_term = jnp.sum(hinged) / (C * (C - 1))
        variance_term = jnp.sum(seghinge_ref[...] * invc_ref[...]) / C
        reg_term = jnp.sum(jnp.sqrt(mun2_row)) / C
        cols = jax.lax.broadcasted_iota(jnp.int32, (1, C), 1)
        count0 = jnp.sum(jnp.where(cols == 0, cnt_ref[...], 0.0))
        m2 = p_total - count0
        denom = jnp.maximum(p2_ref[0, 0] + m2, EPS)
        dice = 2.0 * inter_ref[0, 0] / denom
        instance_term = 1.0 - dice
        loss = (ALPHA * variance_term + BETA * distance_term
                + GAMMA * reg_term + INSTANCE_W * instance_term)
        out_ref[0, 0] = 2.0 * loss


@jax.jit
def _run(emb, tgt):
    p = emb.shape[1]
    nb = p // BLOCK
    out = pl.pallas_call(
        functools.partial(_loss_kernel, float(p)),
        grid=(2, nb),
        in_specs=[
            pl.BlockSpec((16, BLOCK), lambda pp, i: (0, i)),
            pl.BlockSpec((1, BLOCK), lambda pp, i: (0, i)),
        ],
        out_specs=pl.BlockSpec((1, 1), lambda pp, i: (0, 0),
                               memory_space=pltpu.SMEM),
        out_shape=jax.ShapeDtypeStruct((1, 1), jnp.float32),
        scratch_shapes=[
            pltpu.VMEM((18, C), jnp.float32),
            pltpu.VMEM((1, C), jnp.float32),
            pltpu.VMEM((C, 16), jnp.float32),
            pltpu.VMEM((16, C), jnp.float32),
            pltpu.VMEM((C, 18), jnp.bfloat16),
            pltpu.VMEM((1, C), jnp.float32),
            pltpu.VMEM((1, C), jnp.float32),
            pltpu.SMEM((1, 1), jnp.float32),
            pltpu.SMEM((1, 1), jnp.float32),
        ],
    )(emb, tgt)
    return out[0, 0]


def kernel(input_, target):
    n_batches = input_.shape[0]
    emb = input_[n_batches - 1].reshape(16, -1)
    tgt = target[n_batches - 1, 0].reshape(1, -1)
    return _run(emb, tgt) / n_batches


# speedup vs baseline: 1.0364x; 1.0215x over previous
import functools
import math

import jax
import jax.numpy as jnp
from jax.experimental import pallas as pl
from jax.experimental.pallas import tpu as pltpu

DELTA_VAR = 0.5
DELTA_DIST = 2.0
ALPHA = 1.0
BETA = 1.0
GAMMA = 0.001
INSTANCE_W = 1.0
PMAPS_THRESHOLD = 0.9
TWO_SIGMA = DELTA_VAR * DELTA_VAR / -math.log(PMAPS_THRESHOLD)
NEG_INV_TS = -1.0 / TWO_SIGMA
NEG2 = NEG_INV_TS * math.log2(math.e)
C = 64
EPS = 1e-6

BLOCK = 32768

_DN_RHS_T = (((1,), (1,)), ((), ()))
_DN_MATMUL = (((1,), (0,)), ((), ()))


def _dot(a, b, dn):
    return jax.lax.dot_general(
        a, b, dn,
        preferred_element_type=jnp.float32,
        precision=jax.lax.Precision.DEFAULT)


def _loss_kernel(p_total,
                 emb_ref, tgt_ref, out_ref,
                 sumst_ref, cnt_ref, means_ref, meanst_ref, msc_ref,
                 invc_ref, seghinge_ref, inter_ref, p2_ref):
    p_id = pl.program_id(0)
    i_id = pl.program_id(1)
    nblocks = pl.num_programs(1)

    e = emb_ref[...]
    t = tgt_ref[...]
    b = e.shape[1]
    ids = jax.lax.broadcasted_iota(jnp.int32, (C, b), 0)
    ohb = (ids == t).astype(jnp.bfloat16)
    en2n2 = jnp.sum(e * e, axis=0, keepdims=True) * (2.0 * NEG2)
    eaug = jnp.concatenate(
        [e.astype(jnp.bfloat16),
         jnp.ones((1, b), jnp.bfloat16),
         en2n2.astype(jnp.bfloat16)], axis=0)

    @pl.when(jnp.logical_and(p_id == 0, i_id == 0))
    def _init():
        sumst_ref[...] = jnp.zeros_like(sumst_ref)
        seghinge_ref[...] = jnp.zeros_like(seghinge_ref)
        inter_ref[0, 0] = 0.0
        p2_ref[0, 0] = 0.0

    @pl.when(p_id == 0)
    def _pass0():
        sumst_ref[...] += _dot(eaug, ohb, _DN_RHS_T)

    @pl.when(jnp.logical_and(p_id == 1, i_id == 0))
    def _means():
        safe_row = jnp.maximum(sumst_ref[16:17, :], 1.0)
        invc_ref[...] = 1.0 / safe_row
        cnt_ref[...] = sumst_ref[16:17, :]
        meanst_ref[...] = sumst_ref[0:16, :] / safe_row
        ri = jax.lax.broadcasted_iota(jnp.int32, (C, C), 0)
        ci = jax.lax.broadcasted_iota(jnp.int32, (C, C), 1)
        eye = (ri == ci).astype(jnp.float32)
        mc = _dot(eye, meanst_ref[...], _DN_RHS_T)
        means_ref[...] = mc
        mun2 = jnp.sum(mc * mc, axis=1, keepdims=True)
        rows = jax.lax.broadcasted_iota(jnp.int32, (C, 1), 0)
        mun2n = jnp.where(rows == 0, -1e30, mun2 * (2.0 * NEG2))
        msc_ref[...] = jnp.concatenate(
            [mc * (-4.0 * NEG2), mun2n,
             jnp.ones((C, 1), jnp.float32)], axis=1).astype(jnp.bfloat16)

    @pl.when(p_id == 1)
    def _pass1():
        musel = _dot(meanst_ref[...].astype(jnp.bfloat16), ohb,
                     _DN_MATMUL)
        diff = e - musel
        d2sel = jnp.sum(diff * diff, axis=0, keepdims=True)
        hinge = jnp.maximum(jnp.sqrt(d2sel) - DELTA_VAR, 0.0) ** 2
        seghinge_ref[...] += _dot(hinge.astype(jnp.bfloat16), ohb,
                                  _DN_RHS_T)
        q = jnp.exp2(_dot(msc_ref[...], eaug, _DN_MATMUL))
        p2_ref[0, 0] += jnp.sum(q)
        pmsel = jnp.exp2(d2sel * NEG2)
        inter_ref[0, 0] += jnp.sum(jnp.where(t == 0, 0.0, pmsel))

    @pl.when(jnp.logical_and(p_id == 1, i_id == nblocks - 1))
    def _final():
        means = means_ref[...]
        gm = _dot(means, means, _DN_RHS_T)
        ri = jax.lax.broadcasted_iota(jnp.int32, (C, C), 0)
        ci = jax.lax.broadcasted_iota(jnp.int32, (C, C), 1)
        diag = jnp.where(ri == ci, gm, 0.0)
        mun2_row = jnp.sum(diag, axis=0, keepdims=True)
        mun2_col = jnp.sum(diag, axis=1, keepdims=True)
        dd2 = jnp.maximum(mun2_col + mun2_row - 2.0 * gm, 0.0)
        dmat = jnp.sqrt(dd2)
        hinged = jnp.where(
            ri == ci, 0.0,
            jnp.maximum(2.0 * DELTA_DIST - dmat, 0.0) ** 2)
        distance_term = jnp.sum(hinged) / (C * (C - 1))
        variance_term = jnp.sum(seghinge_ref[...] * invc_ref[...]) / C
        reg_term = jnp.sum(jnp.sqrt(mun2_row)) / C
        cols = jax.lax.broadcasted_iota(jnp.int32, (1, C), 1)
        count0 = jnp.sum(jnp.where(cols == 0, cnt_ref[...], 0.0))
        m2 = p_total - count0
        denom = jnp.maximum(p2_ref[0, 0] + m2, EPS)
        dice = 2.0 * inter_ref[0, 0] / denom
        instance_term = 1.0 - dice
        loss = (ALPHA * variance_term + BETA * distance_term
                + GAMMA * reg_term + INSTANCE_W * instance_term)
        out_ref[0, 0] = 2.0 * loss


@jax.jit
def _run(emb, tgt):
    p = emb.shape[1]
    nb = p // BLOCK
    out = pl.pallas_call(
        functools.partial(_loss_kernel, float(p)),
        grid=(2, nb),
        in_specs=[
            pl.BlockSpec((16, BLOCK), lambda pp, i: (0, i)),
            pl.BlockSpec((1, BLOCK), lambda pp, i: (0, i)),
        ],
        out_specs=pl.BlockSpec((1, 1), lambda pp, i: (0, 0),
                               memory_space=pltpu.SMEM),
        out_shape=jax.ShapeDtypeStruct((1, 1), jnp.float32),
        scratch_shapes=[
            pltpu.VMEM((18, C), jnp.float32),
            pltpu.VMEM((1, C), jnp.float32),
            pltpu.VMEM((C, 16), jnp.float32),
            pltpu.VMEM((16, C), jnp.float32),
            pltpu.VMEM((C, 18), jnp.bfloat16),
            pltpu.VMEM((1, C), jnp.float32),
            pltpu.VMEM((1, C), jnp.float32),
            pltpu.SMEM((1, 1), jnp.float32),
            pltpu.SMEM((1, 1), jnp.float32),
        ],
    )(emb, tgt)
    return out[0, 0]


def kernel(input_, target):
    n_batches = input_.shape[0]
    emb = input_[n_batches - 1].reshape(16, -1)
    tgt = target[n_batches - 1, 0].reshape(1, -1)
    return _run(emb, tgt) / n_batches
